# baseline (speedup 1.0000x reference)
import functools

import jax
import jax.numpy as jnp
from jax import lax
from jax.experimental import pallas as pl
from jax.experimental.pallas import tpu as pltpu
from jax.experimental.pallas import tpu_sc as plsc

_NC = 2
_NS = 16
_NW = _NC * _NS
_CHUNK = 128
_HEAVY_CORE = 0
_F32 = jnp.float32
_HI = lax.Precision.HIGHEST


def _silu(v):
    return v / (1.0 + jnp.exp(-v))


def _sc_mesh():
    return plsc.VectorSubcoreMesh(core_axis_name="c", subcore_axis_name="s")


def _sc_gather_combine(t0, t1, idx2d0, idx2d1, nbuf=2, heavy=None):
    chunk = idx2d0.shape[1]
    rows = idx2d0.shape[0]
    chunks = rows // _NW
    per_w = chunks * chunk
    ep = rows * chunk
    w = t0.shape[1]
    wcols = w // 16
    if heavy is None:
        ca = cb = chunks
    else:
        ca = heavy[1] if heavy[0] == 0 else 2 * chunks - heavy[1]
        cb = 2 * chunks - ca
    cmax = max(ca, cb)
    xpad = cmax - min(ca, cb)

    @functools.partial(
        pl.kernel,
        mesh=_sc_mesh(),
        compiler_params=pltpu.CompilerParams(use_tc_tiling_on_sc=False),
        out_type=jax.ShapeDtypeStruct((ep, w), _F32),
        scratch_types=[
            pltpu.VMEM((cmax, chunk), jnp.int32),
            pltpu.VMEM((cmax, chunk), jnp.int32),
            pltpu.VMEM((nbuf, chunk, w), _F32),
            pltpu.VMEM((nbuf, chunk, w), _F32),
        ] + [pltpu.SemaphoreType.DMA] * (3 * nbuf),
    )
    def k(t0_hbm, t1_hbm, i0_hbm, i1_hbm, out_hbm,
          i0v, i1v, buf0, buf1, *sems):
        gsem0 = sems[0:nbuf]
        gsem1 = sems[nbuf:2 * nbuf]
        ssem = sems[2 * nbuf:3 * nbuf]
        cid = lax.axis_index("c")
        sid = lax.axis_index("s")
        my_chunks = jnp.where(cid == 0, ca, cb)
        base_chunk = jnp.where(cid == 0, sid * ca, _NS * ca + sid * cb)
        pltpu.sync_copy(i0_hbm.at[pl.ds(base_chunk, cmax)], i0v)
        pltpu.sync_copy(i1_hbm.at[pl.ds(base_chunk, cmax)], i1v)

        def g0_pair(t, b):
            return (t0_hbm.at[i0v.at[t]], buf0.at[b], gsem0[b])

        def g1_pair(t, b):
            return (t1_hbm.at[i1v.at[t]], buf1.at[b], gsem1[b])

        def s_pair(t, b):
            dst = out_hbm.at[pl.ds((base_chunk + t) * chunk, chunk)]
            return (buf1.at[b], dst, ssem[b])

        def gissue(t, b):
            pltpu.async_copy(*g0_pair(t, b))
            pltpu.async_copy(*g1_pair(t, b))

        def step(t, b, nxt):
            pltpu.make_async_copy(*g0_pair(t, b)).wait()
            pltpu.make_async_copy(*g1_pair(t, b)).wait()

            def combine(r4, carry):
                for u in range(4):
                    r = r4 * 4 + u
                    for c in range(wcols):
                        sl = pl.ds(c * 16, 16)
                        buf1[b, r, sl] = buf0[b, r, sl] + buf1[b, r, sl]
                return carry

            lax.fori_loop(0, chunk // 4, combine, 0)
            pltpu.async_copy(*s_pair(t, b))
            if nxt:
                pltpu.async_copy(*g0_pair(t + nbuf, b))
                pltpu.make_async_copy(*s_pair(t, b)).wait()
                pltpu.async_copy(*g1_pair(t + nbuf, b))
            else:
                pltpu.make_async_copy(*s_pair(t, b)).wait()

        for b in range(nbuf):
            gissue(b, b)

        def main(j, carry):
            t = nbuf * j
            for b in range(nbuf):
                step(t + b, b, True)
            return carry

        lax.fori_loop(0, my_chunks // nbuf - 1, main, 0)
        for b in range(nbuf):
            step(my_chunks - nbuf + b, b, False)

    if xpad:
        zrow = jnp.zeros((xpad, chunk), jnp.int32)
        idx2d0 = jnp.concatenate([idx2d0, zrow])
        idx2d1 = jnp.concatenate([idx2d1, zrow])
    return k(t0, t1, idx2d0, idx2d1)


def _sc_scatter_add(cs, sidx2d, nacc):
    kn = len(cs)
    widths = [c.shape[1] for c in cs]
    rows = sidx2d.shape[0]
    chunks = rows // _NW
    per_w = chunks * _CHUNK
    rows_t = nacc // _NS

    scratch = [pltpu.VMEM((chunks, _CHUNK), jnp.int32)]
    for w in widths:
        scratch += [
            pltpu.VMEM((2, _CHUNK, w), _F32),
            pltpu.VMEM((rows_t, w), _F32),
            pltpu.VMEM_SHARED((nacc, w), _F32),
        ]
    scratch += [pltpu.SemaphoreType.DMA, pltpu.SemaphoreType.DMA,
                pltpu.SemaphoreType.DMA, pltpu.SemaphoreType.DMA]

    @functools.partial(
        pl.kernel,
        mesh=_sc_mesh(),
        compiler_params=pltpu.CompilerParams(use_tc_tiling_on_sc=False),
        out_type=[jax.ShapeDtypeStruct((_NC, nacc, w), _F32) for w in widths],
        scratch_types=scratch,
    )
    def k(*refs):
        c_hbm = refs[0:kn]
        sidx_hbm = refs[kn]
        z_hbm = refs[kn + 1:2 * kn + 1]
        o_hbm = refs[2 * kn + 1:3 * kn + 1]
        sc = refs[3 * kn + 1:]
        idxv = sc[0]
        bufs = [sc[1 + 3 * i] for i in range(kn)]
        wbs = [sc[2 + 3 * i] for i in range(kn)]
        accs = [sc[3 + 3 * i] for i in range(kn)]
        lsem = (sc[1 + 3 * kn], sc[2 + 3 * kn])
        csem = (sc[3 + 3 * kn], sc[4 + 3 * kn])
        cid = lax.axis_index("c")
        sid = lax.axis_index("s")
        wid = sid * _NC + cid

        @pl.when(sid == 0)
        def _init():
            for i in range(kn):
                pltpu.sync_copy(z_hbm[i], accs[i])

        pltpu.sync_copy(sidx_hbm.at[pl.ds(wid * chunks, chunks)], idxv)
        plsc.subcore_barrier()

        def l_pairs(t, b):
            base = wid * per_w + t * _CHUNK
            return tuple((c_hbm[i].at[pl.ds(base, _CHUNK)], bufs[i].at[b],
                          lsem[b]) for i in range(kn))

        def c_pairs(t, b):
            return tuple((bufs[i].at[b], accs[i].at[idxv.at[t]], csem[b])
                         for i in range(kn))

        def lissue(t, b):
            for p in l_pairs(t, b):
                pltpu.async_copy(*p)

        def step(t, b):
            for p in l_pairs(t, b):
                pltpu.make_async_copy(*p).wait()
            for p in c_pairs(t, b):
                pltpu.async_copy(*p, add=True)
            for p in c_pairs(t, b):
                pltpu.make_async_copy(*p).wait()

        lissue(0, 0)
        lissue(1, 1)

        def main(j, carry):
            t = 2 * j
            for b in (0, 1):
                step(t + b, b)
                lissue(t + b + 2, b)
            return carry

        lax.fori_loop(0, chunks // 2 - 1, main, 0)
        for b in (0, 1):
            step(chunks - 2 + b, b)

        plsc.subcore_barrier()

        r0 = sid * rows_t
        for i in range(kn):
            pltpu.sync_copy(accs[i].at[pl.ds(r0, rows_t)], wbs[i])
            pltpu.sync_copy(wbs[i], o_hbm[i].at[cid, pl.ds(r0, rows_t)])

    zeros = [jnp.zeros((nacc, w), _F32) for w in widths]
    out = k(*cs, sidx2d, *zeros)
    return out if isinstance(out, (list, tuple)) else [out]


def _tc_prep0(zc, xt16, emb, w1a_t, w1b_t, b1):
    n = zc.shape[0]
    zmax = emb.shape[0]
    r = 1000
    g = n // r

    def body(z_ref, x_ref, e_ref, wa_ref, wb_ref, b_ref,
             t0_ref, t1_ref, h_ref):
        zcls = lax.broadcasted_iota(jnp.int32, (1, zmax), 1)
        oh = jnp.where(z_ref[...] == zcls, 1.0, 0.0)
        hb = jnp.dot(oh, e_ref[...], preferred_element_type=_F32,
                     precision=_HI)
        h_ref[...] = hb
        t0_ref[:, :128] = (
            jnp.dot(hb, wa_ref[...], preferred_element_type=_F32,
                    precision=_HI) + b_ref[...])
        t0_ref[:, 128:] = -x_ref[...]
        t1_ref[:, :128] = jnp.dot(hb, wb_ref[...], preferred_element_type=_F32,
                                  precision=_HI)
        t1_ref[:, 128:] = x_ref[...]

    return pl.pallas_call(
        body,
        grid=(g,),
        in_specs=[
            pl.BlockSpec((r, 1), lambda i: (i, 0)),
            pl.BlockSpec((r, 16), lambda i: (i, 0)),
            pl.BlockSpec((zmax, 128), lambda i: (0, 0)),
            pl.BlockSpec((128, 128), lambda i: (0, 0)),
            pl.BlockSpec((128, 128), lambda i: (0, 0)),
            pl.BlockSpec((1, 128), lambda i: (0, 0)),
        ],
        out_specs=[
            pl.BlockSpec((r, 144), lambda i: (i, 0)),
            pl.BlockSpec((r, 144), lambda i: (i, 0)),
            pl.BlockSpec((r, 128), lambda i: (i, 0)),
        ],
        out_shape=[
            jax.ShapeDtypeStruct((n, 144), _F32),
            jax.ShapeDtypeStruct((n, 144), _F32),
            jax.ShapeDtypeStruct((n, 128), _F32),
        ],
    )(zc, xt16, emb, w1a_t, w1b_t, b1)


def _tc_edge(gsum, w1c, w2_t, b2, ww1_t, bw1, ww2, bw2, oh3, want_c2):
    ep = gsum.shape[0]
    r = 4096
    g = ep // r

    def body(g_ref, w1c_ref, w2_ref, b2_ref, ww1_ref, bw1_ref,
             ww2_ref, bw2_ref, oh3_ref, c1_ref, *maybe_c2):
        s = g_ref[...]
        pre = s[:, :128]
        t16 = s[:, 128:]
        d = jnp.sqrt(jnp.sum(t16 * t16, axis=1, keepdims=True))
        a1 = _silu(pre + d * w1c_ref[...])
        m = _silu(jnp.dot(a1, w2_ref[...], preferred_element_type=_F32)
                  + b2_ref[...])
        t = _silu(jnp.dot(m, ww1_ref[...], preferred_element_type=_F32)
                  + bw1_ref[...])
        w = jnp.sum(t * ww2_ref[...], axis=1, keepdims=True) + bw2_ref[...]
        c1_ref[...] = t16 * w + oh3_ref[...]
        if maybe_c2:
            maybe_c2[0][...] = m

    out_specs = [pl.BlockSpec((r, 16), lambda i: (i, 0))]
    out_shape = [jax.ShapeDtypeStruct((ep, 16), _F32)]
    if want_c2:
        out_specs.append(pl.BlockSpec((r, 32), lambda i: (i, 0)))
        out_shape.append(jax.ShapeDtypeStruct((ep, 32), _F32))

    return pl.pallas_call(
        body,
        grid=(g,),
        in_specs=[
            pl.BlockSpec((r, 144), lambda i: (i, 0)),
            pl.BlockSpec((1, 128), lambda i: (0, 0)),
            pl.BlockSpec((128, 32), lambda i: (0, 0)),
            pl.BlockSpec((1, 32), lambda i: (0, 0)),
            pl.BlockSpec((32, 32), lambda i: (0, 0)),
            pl.BlockSpec((1, 32), lambda i: (0, 0)),
            pl.BlockSpec((1, 32), lambda i: (0, 0)),
            pl.BlockSpec((1, 1), lambda i: (0, 0)),
            pl.BlockSpec((1, 16), lambda i: (0, 0)),
        ],
        out_specs=out_specs,
        out_shape=out_shape,
    )(gsum, w1c, w2_t, b2, ww1_t, bw1, ww2, bw2, oh3)


def _tc_node_prep(a1, a2, a1b, a2b, h, xt16, wn1a_t, wn1b_t, bn1, wn2_t,
                  bn2, nw1a_t, nw1b_t, nb1):
    n = h.shape[0]
    r = 1000
    g = n // r

    def body(a1_ref, a2_ref, a1b_ref, a2b_ref, h_ref, x_ref, wa_ref, wb_ref,
             b1_ref, w2_ref, b2_ref, nwa_ref, nwb_ref, nb1_ref, t0_ref,
             t1_ref, h_out, x_out):
        s1 = (a1_ref[0] + a1_ref[1]) + (a1b_ref[0] + a1b_ref[1])
        s2 = (a2_ref[0] + a2_ref[1]) + (a2b_ref[0] + a2b_ref[1])
        lane = lax.broadcasted_iota(jnp.int32, (1, 16), 1)
        cnt = jnp.sum(jnp.where(lane == 3, s1, 0.0), axis=1, keepdims=True)
        m_i = s2 / jnp.maximum(cnt, 1.0)
        xn = x_ref[...] + jnp.where(lane < 3, s1, 0.0)
        x_out[...] = xn
        hb = h_ref[...]
        t = (jnp.dot(hb, wa_ref[...], preferred_element_type=_F32)
             + jnp.dot(m_i, wb_ref[...], preferred_element_type=_F32)
             + b1_ref[...])
        hn = hb + (jnp.dot(_silu(t), w2_ref[...], preferred_element_type=_F32)
                   + b2_ref[...])
        h_out[...] = hn
        t0_ref[:, :128] = (
            jnp.dot(hn, nwa_ref[...], preferred_element_type=_F32,
                    precision=_HI) + nb1_ref[...])
        t0_ref[:, 128:] = -xn
        t1_ref[:, :128] = jnp.dot(hn, nwb_ref[...], preferred_element_type=_F32,
                                  precision=_HI)
        t1_ref[:, 128:] = xn

    return pl.pallas_call(
        body,
        grid=(g,),
        in_specs=[
            pl.BlockSpec((2, r, 16), lambda i: (0, i, 0)),
            pl.BlockSpec((2, r, 32), lambda i: (0, i, 0)),
            pl.BlockSpec((2, r, 16), lambda i: (0, i, 0)),
            pl.BlockSpec((2, r, 32), lambda i: (0, i, 0)),
            pl.BlockSpec((r, 128), lambda i: (i, 0)),
            pl.BlockSpec((r, 16), lambda i: (i, 0)),
            pl.BlockSpec((128, 128), lambda i: (0, 0)),
            pl.BlockSpec((32, 128), lambda i: (0, 0)),
            pl.BlockSpec((1, 128), lambda i: (0, 0)),
            pl.BlockSpec((128, 128), lambda i: (0, 0)),
            pl.BlockSpec((1, 128), lambda i: (0, 0)),
            pl.BlockSpec((128, 128), lambda i: (0, 0)),
            pl.BlockSpec((128, 128), lambda i: (0, 0)),
            pl.BlockSpec((1, 128), lambda i: (0, 0)),
        ],
        out_specs=[
            pl.BlockSpec((r, 144), lambda i: (i, 0)),
            pl.BlockSpec((r, 144), lambda i: (i, 0)),
            pl.BlockSpec((r, 128), lambda i: (i, 0)),
            pl.BlockSpec((r, 16), lambda i: (i, 0)),
        ],
        out_shape=[
            jax.ShapeDtypeStruct((n, 144), _F32),
            jax.ShapeDtypeStruct((n, 144), _F32),
            jax.ShapeDtypeStruct((n, 128), _F32),
            jax.ShapeDtypeStruct((n, 16), _F32),
        ],
    )(a1, a2, a1b, a2b, h, xt16, wn1a_t, wn1b_t, bn1, wn2_t, bn2,
      nw1a_t, nw1b_t, nb1)


def _tc_node_final(a1, a1b, xt16):
    n = xt16.shape[0]
    r = 1000
    g = n // r

    def body(a1_ref, a1b_ref, x_ref, x_out):
        s1 = (a1_ref[0] + a1_ref[1]) + (a1b_ref[0] + a1b_ref[1])
        lane = lax.broadcasted_iota(jnp.int32, (1, 16), 1)
        x_out[...] = x_ref[...] + jnp.where(lane < 3, s1, 0.0)

    return pl.pallas_call(
        body,
        grid=(g,),
        in_specs=[
            pl.BlockSpec((2, r, 16), lambda i: (0, i, 0)),
            pl.BlockSpec((2, r, 16), lambda i: (0, i, 0)),
            pl.BlockSpec((r, 16), lambda i: (i, 0)),
        ],
        out_specs=pl.BlockSpec((r, 16), lambda i: (i, 0)),
        out_shape=jax.ShapeDtypeStruct((n, 16), _F32),
    )(a1, a1b, xt16)


def kernel(x, x_thild, z, num_atoms, edges, emb, cov, params):
    del x, num_atoms, cov
    n = x_thild.shape[0]
    e = edges.shape[1]
    nacc = 10240
    step = _NW * _CHUNK * 2
    ep = ((e + step - 1) // step) * step

    e0 = edges[0].astype(jnp.int32)
    e1 = edges[1].astype(jnp.int32)
    pad = ep - e
    eg0 = jnp.concatenate([e0, jnp.zeros((pad,), jnp.int32)]).reshape(-1, 128)
    eg1 = jnp.concatenate([e1, jnp.zeros((pad,), jnp.int32)]).reshape(-1, 128)
    es0 = jnp.concatenate([e0, jnp.full((pad,), n, jnp.int32)]).reshape(-1, 128)

    xt16 = jnp.pad(x_thild, ((0, 0), (0, 13)))
    oh3 = jnp.zeros((1, 16), _F32).at[0, 3].set(1.0)

    p0 = params["layer0"]
    p1 = params["layer1"]
    w1_l0 = p0["edge1"]["W"]
    w1_l1 = p1["edge1"]["W"]

    def edge_args(p):
        return (p["edge1"]["W"][:, 256].reshape(1, 128),
                p["edge2"]["W"].T, p["edge2"]["b"].reshape(1, 32),
                p["w1"]["W"].T, p["w1"]["b"].reshape(1, 32),
                p["w2"]["W"].reshape(1, 32), p["w2"]["b"].reshape(1, 1))

    hg = eg0.shape[0] // 2
    hs = es0.shape[0] // 2
    halves = ((eg0[:hg], eg1[:hg], es0[:hs]),
              (eg0[hg:], eg1[hg:], es0[hs:]))

    t0, t1, h = _tc_prep0(
        z.astype(jnp.int32).reshape(n, 1), xt16, emb,
        w1_l0[:, :128].T, w1_l0[:, 128:256].T,
        p0["edge1"]["b"].reshape(1, 128))
    acc = []
    for g0x, g1x, esx in halves:
        gsum = _sc_gather_combine(t0, t1, g0x, g1x, heavy=(_HEAVY_CORE, 56))
        c1, c2 = _tc_edge(gsum, *edge_args(p0), oh3, want_c2=True)
        acc.append(_sc_scatter_add([c1, c2], esx, nacc))
    t0, t1, h, xt16 = _tc_node_prep(
        acc[0][0], acc[0][1], acc[1][0], acc[1][1], h, xt16,
        p0["node1"]["W"][:, :128].T, p0["node1"]["W"][:, 128:].T,
        p0["node1"]["b"].reshape(1, 128),
        p0["node2"]["W"].T, p0["node2"]["b"].reshape(1, 128),
        w1_l1[:, :128].T, w1_l1[:, 128:256].T,
        p1["edge1"]["b"].reshape(1, 128))

    acc = []
    for g0x, g1x, esx in halves:
        gsum = _sc_gather_combine(t0, t1, g0x, g1x, heavy=(_HEAVY_CORE, 56))
        (c1,) = _tc_edge(gsum, *edge_args(p1), oh3, want_c2=False)
        acc.append(_sc_scatter_add([c1], esx, nacc)[0])
    xt16 = _tc_node_final(acc[0], acc[1], xt16)

    return xt16[:, :3]


# speedup vs baseline: 1.0104x; 1.0104x over previous
import functools

import jax
import jax.numpy as jnp
from jax import lax
from jax.experimental import pallas as pl
from jax.experimental.pallas import tpu as pltpu
from jax.experimental.pallas import tpu_sc as plsc

_NC = 2
_NS = 16
_NW = _NC * _NS
_CHUNK = 128
_HEAVY_CORE = 0
_F32 = jnp.float32
_HI = lax.Precision.HIGHEST


def _silu(v):
    return v / (1.0 + jnp.exp(-v))


def _sc_mesh():
    return plsc.VectorSubcoreMesh(core_axis_name="c", subcore_axis_name="s")


def _sc_gather_combine(t0, t1, idx2d0, idx2d1, nbuf=4, heavy=None):
    chunk = idx2d0.shape[1]
    rows = idx2d0.shape[0]
    chunks = rows // _NW
    per_w = chunks * chunk
    ep = rows * chunk
    w = t0.shape[1]
    wcols = w // 16
    if heavy is None:
        ca = cb = chunks
    else:
        ca = heavy[1] if heavy[0] == 0 else 2 * chunks - heavy[1]
        cb = 2 * chunks - ca
    cmax = max(ca, cb)
    xpad = cmax - min(ca, cb)

    @functools.partial(
        pl.kernel,
        mesh=_sc_mesh(),
        compiler_params=pltpu.CompilerParams(use_tc_tiling_on_sc=False),
        out_type=jax.ShapeDtypeStruct((ep, w), _F32),
        scratch_types=[
            pltpu.VMEM((cmax, chunk), jnp.int32),
            pltpu.VMEM((cmax, chunk), jnp.int32),
            pltpu.VMEM((nbuf, chunk, w), _F32),
            pltpu.VMEM((nbuf, chunk, w), _F32),
        ] + [pltpu.SemaphoreType.DMA] * (3 * nbuf),
    )
    def k(t0_hbm, t1_hbm, i0_hbm, i1_hbm, out_hbm,
          i0v, i1v, buf0, buf1, *sems):
        gsem0 = sems[0:nbuf]
        gsem1 = sems[nbuf:2 * nbuf]
        ssem = sems[2 * nbuf:3 * nbuf]
        cid = lax.axis_index("c")
        sid = lax.axis_index("s")
        my_chunks = jnp.where(cid == 0, ca, cb)
        base_chunk = jnp.where(cid == 0, sid * ca, _NS * ca + sid * cb)
        pltpu.sync_copy(i0_hbm.at[pl.ds(base_chunk, cmax)], i0v)
        pltpu.sync_copy(i1_hbm.at[pl.ds(base_chunk, cmax)], i1v)

        def g0_pair(t, b):
            return (t0_hbm.at[i0v.at[t]], buf0.at[b], gsem0[b])

        def g1_pair(t, b):
            return (t1_hbm.at[i1v.at[t]], buf1.at[b], gsem1[b])

        def s_pair(t, b):
            dst = out_hbm.at[pl.ds((base_chunk + t) * chunk, chunk)]
            return (buf1.at[b], dst, ssem[b])

        def gissue(t, b):
            pltpu.async_copy(*g0_pair(t, b))
            pltpu.async_copy(*g1_pair(t, b))

        def step(t, b, nxt):
            pltpu.make_async_copy(*g0_pair(t, b)).wait()
            pltpu.make_async_copy(*g1_pair(t, b)).wait()

            def combine(r4, carry):
                for u in range(4):
                    r = r4 * 4 + u
                    for c in range(wcols):
                        sl = pl.ds(c * 16, 16)
                        buf1[b, r, sl] = buf0[b, r, sl] + buf1[b, r, sl]
                return carry

            lax.fori_loop(0, chunk // 4, combine, 0)
            pltpu.async_copy(*s_pair(t, b))
            if nxt:
                pltpu.async_copy(*g0_pair(t + nbuf, b))
                pltpu.make_async_copy(*s_pair(t, b)).wait()
                pltpu.async_copy(*g1_pair(t + nbuf, b))
            else:
                pltpu.make_async_copy(*s_pair(t, b)).wait()

        for b in range(nbuf):
            gissue(b, b)

        def main(j, carry):
            t = nbuf * j
            for b in range(nbuf):
                step(t + b, b, True)
            return carry

        lax.fori_loop(0, my_chunks // nbuf - 1, main, 0)
        for b in range(nbuf):
            step(my_chunks - nbuf + b, b, False)

    if xpad:
        zrow = jnp.zeros((xpad, chunk), jnp.int32)
        idx2d0 = jnp.concatenate([idx2d0, zrow])
        idx2d1 = jnp.concatenate([idx2d1, zrow])
    return k(t0, t1, idx2d0, idx2d1)


def _sc_scatter_add(cs, sidx2d, nacc):
    kn = len(cs)
    widths = [c.shape[1] for c in cs]
    rows = sidx2d.shape[0]
    chunks = rows // _NW
    per_w = chunks * _CHUNK
    rows_t = nacc // _NS

    scratch = [pltpu.VMEM((chunks, _CHUNK), jnp.int32)]
    for w in widths:
        scratch += [
            pltpu.VMEM((2, _CHUNK, w), _F32),
            pltpu.VMEM((rows_t, w), _F32),
            pltpu.VMEM_SHARED((nacc, w), _F32),
        ]
    scratch += [pltpu.SemaphoreType.DMA, pltpu.SemaphoreType.DMA,
                pltpu.SemaphoreType.DMA, pltpu.SemaphoreType.DMA]

    @functools.partial(
        pl.kernel,
        mesh=_sc_mesh(),
        compiler_params=pltpu.CompilerParams(use_tc_tiling_on_sc=False),
        out_type=[jax.ShapeDtypeStruct((_NC, nacc, w), _F32) for w in widths],
        scratch_types=scratch,
    )
    def k(*refs):
        c_hbm = refs[0:kn]
        sidx_hbm = refs[kn]
        z_hbm = refs[kn + 1:2 * kn + 1]
        o_hbm = refs[2 * kn + 1:3 * kn + 1]
        sc = refs[3 * kn + 1:]
        idxv = sc[0]
        bufs = [sc[1 + 3 * i] for i in range(kn)]
        wbs = [sc[2 + 3 * i] for i in range(kn)]
        accs = [sc[3 + 3 * i] for i in range(kn)]
        lsem = (sc[1 + 3 * kn], sc[2 + 3 * kn])
        csem = (sc[3 + 3 * kn], sc[4 + 3 * kn])
        cid = lax.axis_index("c")
        sid = lax.axis_index("s")
        wid = sid * _NC + cid

        @pl.when(sid == 0)
        def _init():
            for i in range(kn):
                pltpu.sync_copy(z_hbm[i], accs[i])

        pltpu.sync_copy(sidx_hbm.at[pl.ds(wid * chunks, chunks)], idxv)
        plsc.subcore_barrier()

        def l_pairs(t, b):
            base = wid * per_w + t * _CHUNK
            return tuple((c_hbm[i].at[pl.ds(base, _CHUNK)], bufs[i].at[b],
                          lsem[b]) for i in range(kn))

        def c_pairs(t, b):
            return tuple((bufs[i].at[b], accs[i].at[idxv.at[t]], csem[b])
                         for i in range(kn))

        def lissue(t, b):
            for p in l_pairs(t, b):
                pltpu.async_copy(*p)

        def step(t, b):
            for p in l_pairs(t, b):
                pltpu.make_async_copy(*p).wait()
            for p in c_pairs(t, b):
                pltpu.async_copy(*p, add=True)
            for p in c_pairs(t, b):
                pltpu.make_async_copy(*p).wait()

        lissue(0, 0)
        lissue(1, 1)

        def main(j, carry):
            t = 2 * j
            for b in (0, 1):
                step(t + b, b)
                lissue(t + b + 2, b)
            return carry

        lax.fori_loop(0, chunks // 2 - 1, main, 0)
        for b in (0, 1):
            step(chunks - 2 + b, b)

        plsc.subcore_barrier()

        r0 = sid * rows_t
        for i in range(kn):
            pltpu.sync_copy(accs[i].at[pl.ds(r0, rows_t)], wbs[i])
            pltpu.sync_copy(wbs[i], o_hbm[i].at[cid, pl.ds(r0, rows_t)])

    zeros = [jnp.zeros((nacc, w), _F32) for w in widths]
    out = k(*cs, sidx2d, *zeros)
    return out if isinstance(out, (list, tuple)) else [out]


def _tc_prep0(zc, xt16, emb, w1a_t, w1b_t, b1):
    n = zc.shape[0]
    zmax = emb.shape[0]
    r = 1000
    g = n // r

    def body(z_ref, x_ref, e_ref, wa_ref, wb_ref, b_ref,
             t0_ref, t1_ref, h_ref):
        zcls = lax.broadcasted_iota(jnp.int32, (1, zmax), 1)
        oh = jnp.where(z_ref[...] == zcls, 1.0, 0.0)
        hb = jnp.dot(oh, e_ref[...], preferred_element_type=_F32,
                     precision=_HI)
        h_ref[...] = hb
        t0_ref[:, :128] = (
            jnp.dot(hb, wa_ref[...], preferred_element_type=_F32,
                    precision=_HI) + b_ref[...])
        t0_ref[:, 128:] = -x_ref[...]
        t1_ref[:, :128] = jnp.dot(hb, wb_ref[...], preferred_element_type=_F32,
                                  precision=_HI)
        t1_ref[:, 128:] = x_ref[...]

    return pl.pallas_call(
        body,
        grid=(g,),
        in_specs=[
            pl.BlockSpec((r, 1), lambda i: (i, 0)),
            pl.BlockSpec((r, 16), lambda i: (i, 0)),
            pl.BlockSpec((zmax, 128), lambda i: (0, 0)),
            pl.BlockSpec((128, 128), lambda i: (0, 0)),
            pl.BlockSpec((128, 128), lambda i: (0, 0)),
            pl.BlockSpec((1, 128), lambda i: (0, 0)),
        ],
        out_specs=[
            pl.BlockSpec((r, 144), lambda i: (i, 0)),
            pl.BlockSpec((r, 144), lambda i: (i, 0)),
            pl.BlockSpec((r, 128), lambda i: (i, 0)),
        ],
        out_shape=[
            jax.ShapeDtypeStruct((n, 144), _F32),
            jax.ShapeDtypeStruct((n, 144), _F32),
            jax.ShapeDtypeStruct((n, 128), _F32),
        ],
    )(zc, xt16, emb, w1a_t, w1b_t, b1)


def _tc_edge(gsum, w1c, w2_t, b2, ww1_t, bw1, ww2, bw2, oh3, want_c2):
    ep = gsum.shape[0]
    r = 4096
    g = ep // r

    def body(g_ref, w1c_ref, w2_ref, b2_ref, ww1_ref, bw1_ref,
             ww2_ref, bw2_ref, oh3_ref, c1_ref, *maybe_c2):
        s = g_ref[...]
        pre = s[:, :128]
        t16 = s[:, 128:]
        d = jnp.sqrt(jnp.sum(t16 * t16, axis=1, keepdims=True))
        a1 = _silu(pre + d * w1c_ref[...])
        m = _silu(jnp.dot(a1, w2_ref[...], preferred_element_type=_F32)
                  + b2_ref[...])
        t = _silu(jnp.dot(m, ww1_ref[...], preferred_element_type=_F32)
                  + bw1_ref[...])
        w = jnp.sum(t * ww2_ref[...], axis=1, keepdims=True) + bw2_ref[...]
        c1_ref[...] = t16 * w + oh3_ref[...]
        if maybe_c2:
            maybe_c2[0][...] = m

    out_specs = [pl.BlockSpec((r, 16), lambda i: (i, 0))]
    out_shape = [jax.ShapeDtypeStruct((ep, 16), _F32)]
    if want_c2:
        out_specs.append(pl.BlockSpec((r, 32), lambda i: (i, 0)))
        out_shape.append(jax.ShapeDtypeStruct((ep, 32), _F32))

    return pl.pallas_call(
        body,
        grid=(g,),
        in_specs=[
            pl.BlockSpec((r, 144), lambda i: (i, 0)),
            pl.BlockSpec((1, 128), lambda i: (0, 0)),
            pl.BlockSpec((128, 32), lambda i: (0, 0)),
            pl.BlockSpec((1, 32), lambda i: (0, 0)),
            pl.BlockSpec((32, 32), lambda i: (0, 0)),
            pl.BlockSpec((1, 32), lambda i: (0, 0)),
            pl.BlockSpec((1, 32), lambda i: (0, 0)),
            pl.BlockSpec((1, 1), lambda i: (0, 0)),
            pl.BlockSpec((1, 16), lambda i: (0, 0)),
        ],
        out_specs=out_specs,
        out_shape=out_shape,
    )(gsum, w1c, w2_t, b2, ww1_t, bw1, ww2, bw2, oh3)


def _tc_node_prep(a1, a2, a1b, a2b, h, xt16, wn1a_t, wn1b_t, bn1, wn2_t,
                  bn2, nw1a_t, nw1b_t, nb1):
    n = h.shape[0]
    r = 1000
    g = n // r

    def body(a1_ref, a2_ref, a1b_ref, a2b_ref, h_ref, x_ref, wa_ref, wb_ref,
             b1_ref, w2_ref, b2_ref, nwa_ref, nwb_ref, nb1_ref, t0_ref,
             t1_ref, h_out, x_out):
        s1 = (a1_ref[0] + a1_ref[1]) + (a1b_ref[0] + a1b_ref[1])
        s2 = (a2_ref[0] + a2_ref[1]) + (a2b_ref[0] + a2b_ref[1])
        lane = lax.broadcasted_iota(jnp.int32, (1, 16), 1)
        cnt = jnp.sum(jnp.where(lane == 3, s1, 0.0), axis=1, keepdims=True)
        m_i = s2 / jnp.maximum(cnt, 1.0)
        xn = x_ref[...] + jnp.where(lane < 3, s1, 0.0)
        x_out[...] = xn
        hb = h_ref[...]
        t = (jnp.dot(hb, wa_ref[...], preferred_element_type=_F32)
             + jnp.dot(m_i, wb_ref[...], preferred_element_type=_F32)
             + b1_ref[...])
        hn = hb + (jnp.dot(_silu(t), w2_ref[...], preferred_element_type=_F32)
                   + b2_ref[...])
        h_out[...] = hn
        t0_ref[:, :128] = (
            jnp.dot(hn, nwa_ref[...], preferred_element_type=_F32,
                    precision=_HI) + nb1_ref[...])
        t0_ref[:, 128:] = -xn
        t1_ref[:, :128] = jnp.dot(hn, nwb_ref[...], preferred_element_type=_F32,
                                  precision=_HI)
        t1_ref[:, 128:] = xn

    return pl.pallas_call(
        body,
        grid=(g,),
        in_specs=[
            pl.BlockSpec((2, r, 16), lambda i: (0, i, 0)),
            pl.BlockSpec((2, r, 32), lambda i: (0, i, 0)),
            pl.BlockSpec((2, r, 16), lambda i: (0, i, 0)),
            pl.BlockSpec((2, r, 32), lambda i: (0, i, 0)),
            pl.BlockSpec((r, 128), lambda i: (i, 0)),
            pl.BlockSpec((r, 16), lambda i: (i, 0)),
            pl.BlockSpec((128, 128), lambda i: (0, 0)),
            pl.BlockSpec((32, 128), lambda i: (0, 0)),
            pl.BlockSpec((1, 128), lambda i: (0, 0)),
            pl.BlockSpec((128, 128), lambda i: (0, 0)),
            pl.BlockSpec((1, 128), lambda i: (0, 0)),
            pl.BlockSpec((128, 128), lambda i: (0, 0)),
            pl.BlockSpec((128, 128), lambda i: (0, 0)),
            pl.BlockSpec((1, 128), lambda i: (0, 0)),
        ],
        out_specs=[
            pl.BlockSpec((r, 144), lambda i: (i, 0)),
            pl.BlockSpec((r, 144), lambda i: (i, 0)),
            pl.BlockSpec((r, 128), lambda i: (i, 0)),
            pl.BlockSpec((r, 16), lambda i: (i, 0)),
        ],
        out_shape=[
            jax.ShapeDtypeStruct((n, 144), _F32),
            jax.ShapeDtypeStruct((n, 144), _F32),
            jax.ShapeDtypeStruct((n, 128), _F32),
            jax.ShapeDtypeStruct((n, 16), _F32),
        ],
    )(a1, a2, a1b, a2b, h, xt16, wn1a_t, wn1b_t, bn1, wn2_t, bn2,
      nw1a_t, nw1b_t, nb1)


def _tc_node_final(a1, a1b, xt16):
    n = xt16.shape[0]
    r = 1000
    g = n // r

    def body(a1_ref, a1b_ref, x_ref, x_out):
        s1 = (a1_ref[0] + a1_ref[1]) + (a1b_ref[0] + a1b_ref[1])
        lane = lax.broadcasted_iota(jnp.int32, (1, 16), 1)
        x_out[...] = x_ref[...] + jnp.where(lane < 3, s1, 0.0)

    return pl.pallas_call(
        body,
        grid=(g,),
        in_specs=[
            pl.BlockSpec((2, r, 16), lambda i: (0, i, 0)),
            pl.BlockSpec((2, r, 16), lambda i: (0, i, 0)),
            pl.BlockSpec((r, 16), lambda i: (i, 0)),
        ],
        out_specs=pl.BlockSpec((r, 16), lambda i: (i, 0)),
        out_shape=jax.ShapeDtypeStruct((n, 16), _F32),
    )(a1, a1b, xt16)


def kernel(x, x_thild, z, num_atoms, edges, emb, cov, params):
    del x, num_atoms, cov
    n = x_thild.shape[0]
    e = edges.shape[1]
    nacc = 10240
    step = _NW * _CHUNK * 2
    ep = ((e + step - 1) // step) * step

    e0 = edges[0].astype(jnp.int32)
    e1 = edges[1].astype(jnp.int32)
    pad = ep - e
    eg0 = jnp.concatenate([e0, jnp.zeros((pad,), jnp.int32)]).reshape(-1, 64)
    eg1 = jnp.concatenate([e1, jnp.zeros((pad,), jnp.int32)]).reshape(-1, 64)
    es0 = jnp.concatenate([e0, jnp.full((pad,), n, jnp.int32)]).reshape(-1, 128)

    xt16 = jnp.pad(x_thild, ((0, 0), (0, 13)))
    oh3 = jnp.zeros((1, 16), _F32).at[0, 3].set(1.0)

    p0 = params["layer0"]
    p1 = params["layer1"]
    w1_l0 = p0["edge1"]["W"]
    w1_l1 = p1["edge1"]["W"]

    def edge_args(p):
        return (p["edge1"]["W"][:, 256].reshape(1, 128),
                p["edge2"]["W"].T, p["edge2"]["b"].reshape(1, 32),
                p["w1"]["W"].T, p["w1"]["b"].reshape(1, 32),
                p["w2"]["W"].reshape(1, 32), p["w2"]["b"].reshape(1, 1))

    hg = eg0.shape[0] // 2
    hs = es0.shape[0] // 2
    halves = ((eg0[:hg], eg1[:hg], es0[:hs]),
              (eg0[hg:], eg1[hg:], es0[hs:]))

    t0, t1, h = _tc_prep0(
        z.astype(jnp.int32).reshape(n, 1), xt16, emb,
        w1_l0[:, :128].T, w1_l0[:, 128:256].T,
        p0["edge1"]["b"].reshape(1, 128))
    acc = []
    for g0x, g1x, esx in halves:
        gsum = _sc_gather_combine(t0, t1, g0x, g1x, heavy=None)
        c1, c2 = _tc_edge(gsum, *edge_args(p0), oh3, want_c2=True)
        acc.append(_sc_scatter_add([c1, c2], esx, nacc))
    t0, t1, h, xt16 = _tc_node_prep(
        acc[0][0], acc[0][1], acc[1][0], acc[1][1], h, xt16,
        p0["node1"]["W"][:, :128].T, p0["node1"]["W"][:, 128:].T,
        p0["node1"]["b"].reshape(1, 128),
        p0["node2"]["W"].T, p0["node2"]["b"].reshape(1, 128),
        w1_l1[:, :128].T, w1_l1[:, 128:256].T,
        p1["edge1"]["b"].reshape(1, 128))

    acc = []
    for g0x, g1x, esx in halves:
        gsum = _sc_gather_combine(t0, t1, g0x, g1x, heavy=None)
        (c1,) = _tc_edge(gsum, *edge_args(p1), oh3, want_c2=False)
        acc.append(_sc_scatter_add([c1], esx, nacc)[0])
    xt16 = _tc_node_final(acc[0], acc[1], xt16)

    return xt16[:, :3]


# speedup vs baseline: 1.0304x; 1.0198x over previous
import functools

import jax
import jax.numpy as jnp
from jax import lax
from jax.experimental import pallas as pl
from jax.experimental.pallas import tpu as pltpu
from jax.experimental.pallas import tpu_sc as plsc

_NC = 2
_NS = 16
_NW = _NC * _NS
_CHUNK = 128
_HEAVY_CORE = 0
_F32 = jnp.float32
_HI = lax.Precision.HIGHEST


def _silu(v):
    return v / (1.0 + jnp.exp(-v))


def _sc_mesh():
    return plsc.VectorSubcoreMesh(core_axis_name="c", subcore_axis_name="s")


def _sc_gather_combine(t0, t1, idx2d0, idx2d1, nbuf=4, heavy=None):
    chunk = idx2d0.shape[1]
    rows = idx2d0.shape[0]
    chunks = rows // _NW
    per_w = chunks * chunk
    ep = rows * chunk
    w = t0.shape[1]
    wcols = w // 16
    if heavy is None:
        ca = cb = chunks
    else:
        ca = heavy[1] if heavy[0] == 0 else 2 * chunks - heavy[1]
        cb = 2 * chunks - ca
    cmax = max(ca, cb)
    xpad = cmax - min(ca, cb)

    @functools.partial(
        pl.kernel,
        mesh=_sc_mesh(),
        compiler_params=pltpu.CompilerParams(use_tc_tiling_on_sc=False),
        out_type=jax.ShapeDtypeStruct((ep, w), _F32),
        scratch_types=[
            pltpu.VMEM((cmax, chunk), jnp.int32),
            pltpu.VMEM((cmax, chunk), jnp.int32),
            pltpu.VMEM((nbuf, chunk, w), _F32),
            pltpu.VMEM((nbuf, chunk, w), _F32),
        ] + [pltpu.SemaphoreType.DMA] * (3 * nbuf),
    )
    def k(t0_hbm, t1_hbm, i0_hbm, i1_hbm, out_hbm,
          i0v, i1v, buf0, buf1, *sems):
        gsem0 = sems[0:nbuf]
        gsem1 = sems[nbuf:2 * nbuf]
        ssem = sems[2 * nbuf:3 * nbuf]
        cid = lax.axis_index("c")
        sid = lax.axis_index("s")
        my_chunks = jnp.where(cid == 0, ca, cb)
        base_chunk = jnp.where(cid == 0, sid * ca, _NS * ca + sid * cb)
        pltpu.sync_copy(i0_hbm.at[pl.ds(base_chunk, cmax)], i0v)
        pltpu.sync_copy(i1_hbm.at[pl.ds(base_chunk, cmax)], i1v)

        def g0_pair(t, b):
            return (t0_hbm.at[i0v.at[t]], buf0.at[b], gsem0[b])

        def g1_pair(t, b):
            return (t1_hbm.at[i1v.at[t]], buf1.at[b], gsem1[b])

        def s_pair(t, b):
            dst = out_hbm.at[pl.ds((base_chunk + t) * chunk, chunk)]
            return (buf1.at[b], dst, ssem[b])

        def gissue(t, b):
            pltpu.async_copy(*g0_pair(t, b))
            pltpu.async_copy(*g1_pair(t, b))

        def step(t, b, nxt):
            pltpu.make_async_copy(*g0_pair(t, b)).wait()
            pltpu.make_async_copy(*g1_pair(t, b)).wait()

            def combine(r4, carry):
                for u in range(4):
                    r = r4 * 4 + u
                    for c in range(wcols):
                        sl = pl.ds(c * 16, 16)
                        buf1[b, r, sl] = buf0[b, r, sl] + buf1[b, r, sl]
                return carry

            lax.fori_loop(0, chunk // 4, combine, 0)
            pltpu.async_copy(*s_pair(t, b))
            if nxt:
                pltpu.async_copy(*g0_pair(t + nbuf, b))
                pltpu.make_async_copy(*s_pair(t, b)).wait()
                pltpu.async_copy(*g1_pair(t + nbuf, b))
            else:
                pltpu.make_async_copy(*s_pair(t, b)).wait()

        for b in range(nbuf):
            gissue(b, b)

        def main(j, carry):
            t = nbuf * j
            for b in range(nbuf):
                step(t + b, b, True)
            return carry

        lax.fori_loop(0, my_chunks // nbuf - 1, main, 0)
        for b in range(nbuf):
            step(my_chunks - nbuf + b, b, False)

    if xpad:
        zrow = jnp.zeros((xpad, chunk), jnp.int32)
        idx2d0 = jnp.concatenate([idx2d0, zrow])
        idx2d1 = jnp.concatenate([idx2d1, zrow])
    return k(t0, t1, idx2d0, idx2d1)


def _sc_scatter_add(cs, sidx2d, nacc):
    kn = len(cs)
    widths = [c.shape[1] for c in cs]
    rows = sidx2d.shape[0]
    chunks = rows // _NW
    per_w = chunks * _CHUNK
    rows_t = nacc // _NS

    scratch = [pltpu.VMEM((chunks, _CHUNK), jnp.int32)]
    for w in widths:
        scratch += [
            pltpu.VMEM((2, _CHUNK, w), _F32),
            pltpu.VMEM((rows_t, w), _F32),
            pltpu.VMEM_SHARED((nacc, w), _F32),
        ]
    scratch += [pltpu.SemaphoreType.DMA, pltpu.SemaphoreType.DMA,
                pltpu.SemaphoreType.DMA, pltpu.SemaphoreType.DMA]

    @functools.partial(
        pl.kernel,
        mesh=_sc_mesh(),
        compiler_params=pltpu.CompilerParams(use_tc_tiling_on_sc=False),
        out_type=[jax.ShapeDtypeStruct((_NC, nacc, w), _F32) for w in widths],
        scratch_types=scratch,
    )
    def k(*refs):
        c_hbm = refs[0:kn]
        sidx_hbm = refs[kn]
        z_hbm = refs[kn + 1:2 * kn + 1]
        o_hbm = refs[2 * kn + 1:3 * kn + 1]
        sc = refs[3 * kn + 1:]
        idxv = sc[0]
        bufs = [sc[1 + 3 * i] for i in range(kn)]
        wbs = [sc[2 + 3 * i] for i in range(kn)]
        accs = [sc[3 + 3 * i] for i in range(kn)]
        lsem = (sc[1 + 3 * kn], sc[2 + 3 * kn])
        csem = (sc[3 + 3 * kn], sc[4 + 3 * kn])
        cid = lax.axis_index("c")
        sid = lax.axis_index("s")
        wid = sid * _NC + cid

        @pl.when(sid == 0)
        def _init():
            for i in range(kn):
                pltpu.sync_copy(z_hbm[i], accs[i])

        pltpu.sync_copy(sidx_hbm.at[pl.ds(wid * chunks, chunks)], idxv)
        plsc.subcore_barrier()

        def l_pairs(t, b):
            base = wid * per_w + t * _CHUNK
            return tuple((c_hbm[i].at[pl.ds(base, _CHUNK)], bufs[i].at[b],
                          lsem[b]) for i in range(kn))

        def c_pairs(t, b):
            return tuple((bufs[i].at[b], accs[i].at[idxv.at[t]], csem[b])
                         for i in range(kn))

        def lissue(t, b):
            for p in l_pairs(t, b):
                pltpu.async_copy(*p)

        def step(t, b):
            for p in l_pairs(t, b):
                pltpu.make_async_copy(*p).wait()
            for p in c_pairs(t, b):
                pltpu.async_copy(*p, add=True)
            for p in c_pairs(t, b):
                pltpu.make_async_copy(*p).wait()

        lissue(0, 0)
        lissue(1, 1)

        def main(j, carry):
            t = 2 * j
            for b in (0, 1):
                step(t + b, b)
                lissue(t + b + 2, b)
            return carry

        lax.fori_loop(0, chunks // 2 - 1, main, 0)
        for b in (0, 1):
            step(chunks - 2 + b, b)

        plsc.subcore_barrier()

        r0 = sid * rows_t
        for i in range(kn):
            pltpu.sync_copy(accs[i].at[pl.ds(r0, rows_t)], wbs[i])
            pltpu.sync_copy(wbs[i], o_hbm[i].at[cid, pl.ds(r0, rows_t)])

    zeros = [jnp.zeros((nacc, w), _F32) for w in widths]
    out = k(*cs, sidx2d, *zeros)
    return out if isinstance(out, (list, tuple)) else [out]


def _tc_prep0(zc, xt16, emb, w1a_t, w1b_t, b1):
    n = zc.shape[0]
    zmax = emb.shape[0]
    r = 1000
    g = n // r

    def body(z_ref, x_ref, e_ref, wa_ref, wb_ref, b_ref,
             t0_ref, t1_ref, h_ref):
        zcls = lax.broadcasted_iota(jnp.int32, (1, zmax), 1)
        oh = jnp.where(z_ref[...] == zcls, 1.0, 0.0)
        hb = jnp.dot(oh, e_ref[...], preferred_element_type=_F32,
                     precision=_HI)
        h_ref[...] = hb
        t0_ref[:, :128] = (
            jnp.dot(hb, wa_ref[...], preferred_element_type=_F32,
                    precision=_HI) + b_ref[...])
        t0_ref[:, 128:] = -x_ref[...]
        t1_ref[:, :128] = jnp.dot(hb, wb_ref[...], preferred_element_type=_F32,
                                  precision=_HI)
        t1_ref[:, 128:] = x_ref[...]

    return pl.pallas_call(
        body,
        grid=(g,),
        in_specs=[
            pl.BlockSpec((r, 1), lambda i: (i, 0)),
            pl.BlockSpec((r, 16), lambda i: (i, 0)),
            pl.BlockSpec((zmax, 128), lambda i: (0, 0)),
            pl.BlockSpec((128, 128), lambda i: (0, 0)),
            pl.BlockSpec((128, 128), lambda i: (0, 0)),
            pl.BlockSpec((1, 128), lambda i: (0, 0)),
        ],
        out_specs=[
            pl.BlockSpec((r, 144), lambda i: (i, 0)),
            pl.BlockSpec((r, 144), lambda i: (i, 0)),
            pl.BlockSpec((r, 128), lambda i: (i, 0)),
        ],
        out_shape=[
            jax.ShapeDtypeStruct((n, 144), _F32),
            jax.ShapeDtypeStruct((n, 144), _F32),
            jax.ShapeDtypeStruct((n, 128), _F32),
        ],
    )(zc, xt16, emb, w1a_t, w1b_t, b1)


def _tc_edge(gsum, w1c, w2_t, b2, ww1_t, bw1, ww2, bw2, oh3, want_c2):
    ep = gsum.shape[0]
    r = 4096
    g = ep // r

    def body(g_ref, w1c_ref, w2_ref, b2_ref, ww1_ref, bw1_ref,
             ww2_ref, bw2_ref, oh3_ref, c1_ref, *maybe_c2):
        s = g_ref[...]
        pre = s[:, :128]
        t16 = s[:, 128:]
        d = jnp.sqrt(jnp.sum(t16 * t16, axis=1, keepdims=True))
        a1 = _silu(pre + d * w1c_ref[...])
        m = _silu(jnp.dot(a1, w2_ref[...], preferred_element_type=_F32)
                  + b2_ref[...])
        t = _silu(jnp.dot(m, ww1_ref[...], preferred_element_type=_F32)
                  + bw1_ref[...])
        w = jnp.sum(t * ww2_ref[...], axis=1, keepdims=True) + bw2_ref[...]
        c1_ref[...] = t16 * w + oh3_ref[...]
        if maybe_c2:
            maybe_c2[0][...] = m

    out_specs = [pl.BlockSpec((r, 16), lambda i: (i, 0))]
    out_shape = [jax.ShapeDtypeStruct((ep, 16), _F32)]
    if want_c2:
        out_specs.append(pl.BlockSpec((r, 32), lambda i: (i, 0)))
        out_shape.append(jax.ShapeDtypeStruct((ep, 32), _F32))

    return pl.pallas_call(
        body,
        grid=(g,),
        in_specs=[
            pl.BlockSpec((r, 144), lambda i: (i, 0)),
            pl.BlockSpec((1, 128), lambda i: (0, 0)),
            pl.BlockSpec((128, 32), lambda i: (0, 0)),
            pl.BlockSpec((1, 32), lambda i: (0, 0)),
            pl.BlockSpec((32, 32), lambda i: (0, 0)),
            pl.BlockSpec((1, 32), lambda i: (0, 0)),
            pl.BlockSpec((1, 32), lambda i: (0, 0)),
            pl.BlockSpec((1, 1), lambda i: (0, 0)),
            pl.BlockSpec((1, 16), lambda i: (0, 0)),
        ],
        out_specs=out_specs,
        out_shape=out_shape,
    )(gsum, w1c, w2_t, b2, ww1_t, bw1, ww2, bw2, oh3)


def _tc_node_prep(a1, a2, a1b, a2b, h, xt16, wn1a_t, wn1b_t, bn1, wn2_t,
                  bn2, nw1a_t, nw1b_t, nb1):
    n = h.shape[0]
    r = 1000
    g = n // r

    def body(a1_ref, a2_ref, a1b_ref, a2b_ref, h_ref, x_ref, wa_ref, wb_ref,
             b1_ref, w2_ref, b2_ref, nwa_ref, nwb_ref, nb1_ref, t0_ref,
             t1_ref, h_out, x_out):
        s1 = (a1_ref[0] + a1_ref[1]) + (a1b_ref[0] + a1b_ref[1])
        s2 = (a2_ref[0] + a2_ref[1]) + (a2b_ref[0] + a2b_ref[1])
        lane = lax.broadcasted_iota(jnp.int32, (1, 16), 1)
        cnt = jnp.sum(jnp.where(lane == 3, s1, 0.0), axis=1, keepdims=True)
        m_i = s2 / jnp.maximum(cnt, 1.0)
        xn = x_ref[...] + jnp.where(lane < 3, s1, 0.0)
        x_out[...] = xn
        hb = h_ref[...]
        t = (jnp.dot(hb, wa_ref[...], preferred_element_type=_F32)
             + jnp.dot(m_i, wb_ref[...], preferred_element_type=_F32)
             + b1_ref[...])
        hn = hb + (jnp.dot(_silu(t), w2_ref[...], preferred_element_type=_F32)
                   + b2_ref[...])
        h_out[...] = hn
        t0_ref[:, :128] = (
            jnp.dot(hn, nwa_ref[...], preferred_element_type=_F32,
                    precision=_HI) + nb1_ref[...])
        t0_ref[:, 128:] = -xn
        t1_ref[:, :128] = jnp.dot(hn, nwb_ref[...], preferred_element_type=_F32,
                                  precision=_HI)
        t1_ref[:, 128:] = xn

    return pl.pallas_call(
        body,
        grid=(g,),
        in_specs=[
            pl.BlockSpec((2, r, 16), lambda i: (0, i, 0)),
            pl.BlockSpec((2, r, 32), lambda i: (0, i, 0)),
            pl.BlockSpec((2, r, 16), lambda i: (0, i, 0)),
            pl.BlockSpec((2, r, 32), lambda i: (0, i, 0)),
            pl.BlockSpec((r, 128), lambda i: (i, 0)),
            pl.BlockSpec((r, 16), lambda i: (i, 0)),
            pl.BlockSpec((128, 128), lambda i: (0, 0)),
            pl.BlockSpec((32, 128), lambda i: (0, 0)),
            pl.BlockSpec((1, 128), lambda i: (0, 0)),
            pl.BlockSpec((128, 128), lambda i: (0, 0)),
            pl.BlockSpec((1, 128), lambda i: (0, 0)),
            pl.BlockSpec((128, 128), lambda i: (0, 0)),
            pl.BlockSpec((128, 128), lambda i: (0, 0)),
            pl.BlockSpec((1, 128), lambda i: (0, 0)),
        ],
        out_specs=[
            pl.BlockSpec((r, 144), lambda i: (i, 0)),
            pl.BlockSpec((r, 144), lambda i: (i, 0)),
            pl.BlockSpec((r, 128), lambda i: (i, 0)),
            pl.BlockSpec((r, 16), lambda i: (i, 0)),
        ],
        out_shape=[
            jax.ShapeDtypeStruct((n, 144), _F32),
            jax.ShapeDtypeStruct((n, 144), _F32),
            jax.ShapeDtypeStruct((n, 128), _F32),
            jax.ShapeDtypeStruct((n, 16), _F32),
        ],
    )(a1, a2, a1b, a2b, h, xt16, wn1a_t, wn1b_t, bn1, wn2_t, bn2,
      nw1a_t, nw1b_t, nb1)


def _tc_node_final(a1, a1b, xt16):
    n = xt16.shape[0]
    r = 1000
    g = n // r

    def body(a1_ref, a1b_ref, x_ref, x_out):
        s1 = (a1_ref[0] + a1_ref[1]) + (a1b_ref[0] + a1b_ref[1])
        lane = lax.broadcasted_iota(jnp.int32, (1, 16), 1)
        x_out[...] = x_ref[...] + jnp.where(lane < 3, s1, 0.0)

    return pl.pallas_call(
        body,
        grid=(g,),
        in_specs=[
            pl.BlockSpec((2, r, 16), lambda i: (0, i, 0)),
            pl.BlockSpec((2, r, 16), lambda i: (0, i, 0)),
            pl.BlockSpec((r, 16), lambda i: (i, 0)),
        ],
        out_specs=pl.BlockSpec((r, 16), lambda i: (i, 0)),
        out_shape=jax.ShapeDtypeStruct((n, 16), _F32),
    )(a1, a1b, xt16)


def kernel(x, x_thild, z, num_atoms, edges, emb, cov, params):
    del x, num_atoms, cov
    n = x_thild.shape[0]
    e = edges.shape[1]
    nacc = 10240
    step = _NW * _CHUNK * 2
    ep = ((e + step - 1) // step) * step

    e0 = edges[0].astype(jnp.int32)
    e1 = edges[1].astype(jnp.int32)
    pad = ep - e
    eg0 = jnp.concatenate([e0, jnp.zeros((pad,), jnp.int32)]).reshape(-1, 64)
    eg1 = jnp.concatenate([e1, jnp.zeros((pad,), jnp.int32)]).reshape(-1, 64)
    es0 = jnp.concatenate([e0, jnp.full((pad,), n, jnp.int32)]).reshape(-1, 128)

    xt16 = jnp.pad(x_thild, ((0, 0), (0, 13)))
    oh3 = jnp.zeros((1, 16), _F32).at[0, 3].set(1.0)

    p0 = params["layer0"]
    p1 = params["layer1"]
    w1_l0 = p0["edge1"]["W"]
    w1_l1 = p1["edge1"]["W"]

    def edge_args(p):
        return (p["edge1"]["W"][:, 256].reshape(1, 128),
                p["edge2"]["W"].T, p["edge2"]["b"].reshape(1, 32),
                p["w1"]["W"].T, p["w1"]["b"].reshape(1, 32),
                p["w2"]["W"].reshape(1, 32), p["w2"]["b"].reshape(1, 1))

    hg = eg0.shape[0] // 2
    hs = es0.shape[0] // 2
    halves = ((eg0[:hg], eg1[:hg], es0[:hs]),
              (eg0[hg:], eg1[hg:], es0[hs:]))

    t0, t1, h = _tc_prep0(
        z.astype(jnp.int32).reshape(n, 1), xt16, emb,
        w1_l0[:, :128].T, w1_l0[:, 128:256].T,
        p0["edge1"]["b"].reshape(1, 128))
    acc = []
    for g0x, g1x, esx in halves:
        gsum = _sc_gather_combine(t0, t1, g0x, g1x, heavy=(_HEAVY_CORE, 116))
        c1, c2 = _tc_edge(gsum, *edge_args(p0), oh3, want_c2=True)
        acc.append(_sc_scatter_add([c1, c2], esx, nacc))
    t0, t1, h, xt16 = _tc_node_prep(
        acc[0][0], acc[0][1], acc[1][0], acc[1][1], h, xt16,
        p0["node1"]["W"][:, :128].T, p0["node1"]["W"][:, 128:].T,
        p0["node1"]["b"].reshape(1, 128),
        p0["node2"]["W"].T, p0["node2"]["b"].reshape(1, 128),
        w1_l1[:, :128].T, w1_l1[:, 128:256].T,
        p1["edge1"]["b"].reshape(1, 128))

    acc = []
    for g0x, g1x, esx in halves:
        gsum = _sc_gather_combine(t0, t1, g0x, g1x, heavy=(_HEAVY_CORE, 116))
        (c1,) = _tc_edge(gsum, *edge_args(p1), oh3, want_c2=False)
        acc.append(_sc_scatter_add([c1], esx, nacc)[0])
    xt16 = _tc_node_final(acc[0], acc[1], xt16)

    return xt16[:, :3]
